# Initial kernel scaffold; baseline (speedup 1.0000x reference)
import jax
import jax.numpy as jnp
from jax.experimental import pallas as pl


def kernel(x, edge_index):
    raise NotImplementedError("write your pallas kernel here")


# speedup vs baseline: 2.9747x; 2.9747x over previous
import dataclasses
import functools

import jax
import jax.numpy as jnp
from jax import lax
from jax.experimental import pallas as pl
from jax.experimental.pallas import tpu as pltpu
from jax.experimental.pallas import tpu_sc as plsc

N_NODES = 10000
N_HE = 10000
NNZ = 320000
D = 128

NC = 2
NS = 16
NW = NC * NS
L = 16
CHUNK = 128
NCHUNK = 80
GRP = 16
NGRP = NCHUNK // GRP
NNZ_PAD = NW * NCHUNK * CHUNK
ACC_ROWS = 10240
SCRAP = 10000
ROWS_PER_TILE = ACC_ROWS // NS
HR = ACC_ROWS // D
_ZSTEPS = ROWS_PER_TILE // CHUNK

_mesh = plsc.VectorSubcoreMesh(core_axis_name="c", subcore_axis_name="s")

_cp_no_layout = pltpu.CompilerParams()
if "needs_layout_passes" in pltpu.CompilerParams.__dataclass_fields__:
    _cp_no_layout = dataclasses.replace(_cp_no_layout, needs_layout_passes=False)


@functools.partial(
    pl.kernel,
    out_type=(
        jax.ShapeDtypeStruct((NW, HR, D), jnp.float32),
        jax.ShapeDtypeStruct((NW, HR, D), jnp.float32),
    ),
    mesh=_mesh,
    compiler_params=_cp_no_layout,
    scratch_types=[
        pltpu.VMEM((NCHUNK, CHUNK), jnp.int32),
        pltpu.VMEM((NCHUNK, CHUNK), jnp.int32),
        pltpu.VMEM((HR, D), jnp.float32),
        pltpu.VMEM((HR, D), jnp.float32),
    ],
)
def _sc_counts(es_hbm, vs_hbm, zh_hbm,
               ce_hbm, cv_hbm,
               e_v, v_v, he_v, hv_v):
    c = lax.axis_index("c")
    s = lax.axis_index("s")
    wid = s * NC + c

    pltpu.sync_copy(es_hbm.at[wid], e_v)
    pltpu.sync_copy(vs_hbm.at[wid], v_v)
    pltpu.sync_copy(zh_hbm, he_v)
    pltpu.sync_copy(zh_hbm, hv_v)

    ones = jnp.full((L,), 1.0, jnp.float32)

    @pl.loop(0, NCHUNK)
    def _(j):
        @pl.loop(0, CHUNK // L)
        def _(l):
            ide = e_v[j, pl.ds(l * L, L)]
            plsc.addupdate_scatter(
                he_v, [lax.shift_right_logical(ide, 7),
                       lax.bitwise_and(ide, 127)], ones)
            idv = v_v[j, pl.ds(l * L, L)]
            plsc.addupdate_scatter(
                hv_v, [lax.shift_right_logical(idv, 7),
                       lax.bitwise_and(idv, 127)], ones)

    pltpu.sync_copy(he_v, ce_hbm.at[wid])
    pltpu.sync_copy(hv_v, cv_hbm.at[wid])


@functools.partial(
    pl.kernel,
    out_type=jax.ShapeDtypeStruct((NC, ACC_ROWS, D), jnp.float32),
    mesh=_mesh,
    scratch_types=[
        pltpu.VMEM((GRP, CHUNK), jnp.int32),
        pltpu.VMEM((GRP, CHUNK), jnp.int32),
        pltpu.VMEM((CHUNK, D), jnp.float32),
        pltpu.VMEM_SHARED((ACC_ROWS, D), jnp.float32),
    ],
)
def _sc_pass(src_hbm, g_hbm, s_hbm, zd_hbm,
             part_hbm,
             g_v, s_v, rows_v, acc_s):
    c = lax.axis_index("c")
    s = lax.axis_index("s")
    wid = s * NC + c
    base = s * ROWS_PER_TILE

    pltpu.sync_copy(zd_hbm, rows_v)

    @pl.loop(0, _ZSTEPS)
    def _(j):
        pltpu.sync_copy(rows_v, acc_s.at[pl.ds(base + j * CHUNK, CHUNK)])

    plsc.subcore_barrier()

    @pl.loop(0, NGRP)
    def _(g):
        pltpu.sync_copy(g_hbm.at[wid, pl.ds(g * GRP, GRP)], g_v)
        pltpu.sync_copy(s_hbm.at[wid, pl.ds(g * GRP, GRP)], s_v)

        @pl.loop(0, GRP)
        def _(j):
            pltpu.sync_copy(src_hbm.at[g_v.at[j]], rows_v)
            pltpu.sync_copy(rows_v, acc_s.at[s_v.at[j]], add=True)

    plsc.subcore_barrier()

    @pl.loop(0, _ZSTEPS)
    def _(j):
        r0 = base + j * CHUNK
        pltpu.sync_copy(acc_s.at[pl.ds(r0, CHUNK)], rows_v)
        pltpu.sync_copy(rows_v, part_hbm.at[c, pl.ds(r0, CHUNK)])


_BLK = 512
_NBLK = ACC_ROWS // _BLK
_HB = _BLK // D


def _norm_body(p_ref, c_ref, o_ref):
    ps = p_ref[0] + p_ref[1]
    cnt = jnp.sum(c_ref[...], axis=1)
    inv = jnp.where(cnt > 0, 1.0 / cnt, 0.0)
    inv_t = inv.T
    inv_col = jnp.concatenate(
        [inv_t[:, g:g + 1] for g in range(_HB)], axis=0)
    o_ref[...] = ps * inv_col


def _combine_normalize(partials, counts):
    return pl.pallas_call(
        _norm_body,
        grid=(_NBLK,),
        in_specs=[
            pl.BlockSpec((NC, _BLK, D), lambda i: (0, i, 0)),
            pl.BlockSpec((_HB, NW, D), lambda i: (i, 0, 0)),
        ],
        out_specs=pl.BlockSpec((_BLK, D), lambda i: (i, 0)),
        out_shape=jax.ShapeDtypeStruct((ACC_ROWS, D), jnp.float32),
    )(partials, counts)


def kernel(x, edge_index):
    V = edge_index[0].astype(jnp.int32)
    E = edge_index[1].astype(jnp.int32)
    pad = NNZ_PAD - NNZ

    z_pad = jnp.zeros((pad,), jnp.int32)
    s_pad = jnp.full((pad,), SCRAP, jnp.int32)
    vg = jnp.concatenate([V, z_pad]).reshape(NW, NCHUNK, CHUNK)
    eg = jnp.concatenate([E, z_pad]).reshape(NW, NCHUNK, CHUNK)
    es = jnp.concatenate([E, s_pad]).reshape(NW, NCHUNK, CHUNK)
    vs = jnp.concatenate([V, s_pad]).reshape(NW, NCHUNK, CHUNK)

    zd = jnp.zeros((CHUNK, D), jnp.float32)
    zh = jnp.zeros((HR, D), jnp.float32)

    ce, cv = _sc_counts(es, vs, zh)
    ce = jnp.transpose(ce, (1, 0, 2))
    cv = jnp.transpose(cv, (1, 0, 2))
    phe = _sc_pass(x, vg, es, zd)
    he_feat = _combine_normalize(phe, ce)
    pout = _sc_pass(he_feat, eg, vs, zd)
    out = _combine_normalize(pout, cv)
    return out[:N_NODES]


# speedup vs baseline: 3.0454x; 1.0238x over previous
import dataclasses
import functools

import jax
import jax.numpy as jnp
from jax import lax
from jax.experimental import pallas as pl
from jax.experimental.pallas import tpu as pltpu
from jax.experimental.pallas import tpu_sc as plsc

N_NODES = 10000
N_HE = 10000
NNZ = 320000
D = 128

NC = 2
NS = 16
NW = NC * NS
L = 16
CHUNK = 128
NCHUNK = 80
GRP = 16
NGRP = NCHUNK // GRP
NNZ_PAD = NW * NCHUNK * CHUNK
ACC_ROWS = 10240
SCRAP = 10000
ROWS_PER_TILE = ACC_ROWS // NS
HR = ACC_ROWS // D
_ZSTEPS = ROWS_PER_TILE // CHUNK

_mesh = plsc.VectorSubcoreMesh(core_axis_name="c", subcore_axis_name="s")

_cp_no_layout = pltpu.CompilerParams()
if "needs_layout_passes" in pltpu.CompilerParams.__dataclass_fields__:
    _cp_no_layout = dataclasses.replace(_cp_no_layout, needs_layout_passes=False)


@functools.partial(
    pl.kernel,
    out_type=(
        jax.ShapeDtypeStruct((NW, HR, D), jnp.float32),
        jax.ShapeDtypeStruct((NW, HR, D), jnp.float32),
    ),
    mesh=_mesh,
    compiler_params=_cp_no_layout,
    scratch_types=[
        pltpu.VMEM((NCHUNK, CHUNK), jnp.int32),
        pltpu.VMEM((NCHUNK, CHUNK), jnp.int32),
        pltpu.VMEM((HR, D), jnp.float32),
        pltpu.VMEM((HR, D), jnp.float32),
    ],
)
def _sc_counts(es_hbm, vs_hbm, zh_hbm,
               ce_hbm, cv_hbm,
               e_v, v_v, he_v, hv_v):
    c = lax.axis_index("c")
    s = lax.axis_index("s")
    wid = s * NC + c

    pltpu.sync_copy(es_hbm.at[wid], e_v)
    pltpu.sync_copy(vs_hbm.at[wid], v_v)
    pltpu.sync_copy(zh_hbm, he_v)
    pltpu.sync_copy(zh_hbm, hv_v)

    ones = jnp.full((L,), 1.0, jnp.float32)

    @pl.loop(0, NCHUNK)
    def _(j):
        @pl.loop(0, CHUNK // L)
        def _(l):
            ide = e_v[j, pl.ds(l * L, L)]
            plsc.addupdate_scatter(
                he_v, [lax.shift_right_logical(ide, 7),
                       lax.bitwise_and(ide, 127)], ones)
            idv = v_v[j, pl.ds(l * L, L)]
            plsc.addupdate_scatter(
                hv_v, [lax.shift_right_logical(idv, 7),
                       lax.bitwise_and(idv, 127)], ones)

    pltpu.sync_copy(he_v, ce_hbm.at[wid])
    pltpu.sync_copy(hv_v, cv_hbm.at[wid])


@functools.partial(
    pl.kernel,
    out_type=jax.ShapeDtypeStruct((NC, ACC_ROWS, D), jnp.float32),
    mesh=_mesh,
    scratch_types=[
        pltpu.VMEM((GRP, CHUNK), jnp.int32),
        pltpu.VMEM((GRP, CHUNK), jnp.int32),
        pltpu.VMEM((CHUNK, D), jnp.float32),
        pltpu.VMEM((CHUNK, D), jnp.float32),
        pltpu.SemaphoreType.DMA,
        pltpu.SemaphoreType.DMA,
        pltpu.VMEM_SHARED((ACC_ROWS, D), jnp.float32),
    ],
)
def _sc_pass(src_hbm, g_hbm, s_hbm, zd_hbm,
             part_hbm,
             g_v, s_v, rows_a, rows_b, sem_a, sem_b, acc_s):
    c = lax.axis_index("c")
    s = lax.axis_index("s")
    wid = s * NC + c
    base = s * ROWS_PER_TILE

    pltpu.sync_copy(zd_hbm, rows_a)

    @pl.loop(0, _ZSTEPS)
    def _(j):
        pltpu.sync_copy(rows_a, acc_s.at[pl.ds(base + j * CHUNK, CHUNK)])

    plsc.subcore_barrier()

    @pl.loop(0, NGRP)
    def _(g):
        pltpu.sync_copy(g_hbm.at[wid, pl.ds(g * GRP, GRP)], g_v)
        pltpu.sync_copy(s_hbm.at[wid, pl.ds(g * GRP, GRP)], s_v)

        @pl.loop(0, GRP // 2)
        def _(p):
            cp_a = pltpu.async_copy(src_hbm.at[g_v.at[2 * p]], rows_a, sem_a)
            cp_b = pltpu.async_copy(src_hbm.at[g_v.at[2 * p + 1]], rows_b,
                                    sem_b)
            cp_a.wait()
            pltpu.sync_copy(rows_a, acc_s.at[s_v.at[2 * p]], add=True)
            cp_b.wait()
            pltpu.sync_copy(rows_b, acc_s.at[s_v.at[2 * p + 1]], add=True)

    plsc.subcore_barrier()

    @pl.loop(0, _ZSTEPS)
    def _(j):
        r0 = base + j * CHUNK
        pltpu.sync_copy(acc_s.at[pl.ds(r0, CHUNK)], rows_a)
        pltpu.sync_copy(rows_a, part_hbm.at[c, pl.ds(r0, CHUNK)])


_BLK = 512
_NBLK = ACC_ROWS // _BLK
_HB = _BLK // D


def _norm_body(p_ref, c_ref, o_ref):
    ps = p_ref[0] + p_ref[1]
    cnt = jnp.sum(c_ref[...], axis=1)
    inv = jnp.where(cnt > 0, 1.0 / cnt, 0.0)
    inv_t = inv.T
    inv_col = jnp.concatenate(
        [inv_t[:, g:g + 1] for g in range(_HB)], axis=0)
    o_ref[...] = ps * inv_col


def _combine_normalize(partials, counts):
    return pl.pallas_call(
        _norm_body,
        grid=(_NBLK,),
        in_specs=[
            pl.BlockSpec((NC, _BLK, D), lambda i: (0, i, 0)),
            pl.BlockSpec((_HB, NW, D), lambda i: (i, 0, 0)),
        ],
        out_specs=pl.BlockSpec((_BLK, D), lambda i: (i, 0)),
        out_shape=jax.ShapeDtypeStruct((ACC_ROWS, D), jnp.float32),
    )(partials, counts)


def kernel(x, edge_index):
    V = edge_index[0].astype(jnp.int32)
    E = edge_index[1].astype(jnp.int32)
    pad = NNZ_PAD - NNZ

    z_pad = jnp.zeros((pad,), jnp.int32)
    s_pad = jnp.full((pad,), SCRAP, jnp.int32)
    vg = jnp.concatenate([V, z_pad]).reshape(NW, NCHUNK, CHUNK)
    eg = jnp.concatenate([E, z_pad]).reshape(NW, NCHUNK, CHUNK)
    es = jnp.concatenate([E, s_pad]).reshape(NW, NCHUNK, CHUNK)
    vs = jnp.concatenate([V, s_pad]).reshape(NW, NCHUNK, CHUNK)

    zd = jnp.zeros((CHUNK, D), jnp.float32)
    zh = jnp.zeros((HR, D), jnp.float32)

    ce, cv = _sc_counts(es, vs, zh)
    ce = jnp.transpose(ce, (1, 0, 2))
    cv = jnp.transpose(cv, (1, 0, 2))
    phe = _sc_pass(x, vg, es, zd)
    he_feat = _combine_normalize(phe, ce)
    pout = _sc_pass(he_feat, eg, vs, zd)
    out = _combine_normalize(pout, cv)
    return out[:N_NODES]
